# baseline (speedup 1.0000x reference)
import jax
import jax.numpy as jnp
from jax.experimental import pallas as pl
from jax.experimental.pallas import tpu as pltpu

H, CH, W = 128, 64, 256
NE2 = 8
NE3 = 64
O1 = 8
O2 = 12
HID = 32


def _leaky(x):
    return jnp.where(x > 0, x, 0.01 * x)


def _argmax0(a, n):
    mx = jnp.max(a, axis=0)
    iota = jax.lax.broadcasted_iota(jnp.int32, a.shape, 0)
    cand = jnp.where(a == mx[None, :], iota, n)
    return jnp.min(cand, axis=0).astype(jnp.int32)


def _mm(w, hm):
    return jax.lax.dot_general(
        w, hm, (((1,), (0,)), ((), ())), preferred_element_type=jnp.float32)


def _khatri_rao(h, m, ne, d):
    return jnp.concatenate([(h[None] * m[:, None, :]).reshape(ne * d, W), m],
                           axis=0)


def _line_kernel(x_ref, w10, b10, w11, b11, w12, b12,
                 w20, w21, w22, w30, w31, w32, out_ref):
    X = x_ref[0]

    h = _leaky(_mm(w10[0], X) + b10[0])
    h = _leaky(_mm(w11[0], h) + b11[0])
    s1 = _mm(w12[0], h) + b12[0]
    inds1 = _argmax0(s1, O1)

    e2 = jax.lax.broadcasted_iota(jnp.int32, (NE2, W), 0)
    m2 = (e2 == inds1[None, :]).astype(jnp.float32)
    h = _leaky(_mm(w20[0], _khatri_rao(X, m2, NE2, CH)))
    h = _leaky(_mm(w21[0], _khatri_rao(h, m2, NE2, HID)))
    s2 = _mm(w22[0], _khatri_rao(h, m2, NE2, HID))
    inds2 = _argmax0(s2, O2)

    inds12_raw = inds1 * NE2 + inds2 - 2
    inds12 = jnp.clip(inds12_raw, 0, NE3 - 1)

    e3 = jax.lax.broadcasted_iota(jnp.int32, (NE3, W), 0)
    m3 = (e3 == inds12[None, :]).astype(jnp.float32)
    h = _leaky(_mm(w30[0], _khatri_rao(X, m3, NE3, CH)))
    h = _leaky(_mm(w31[0], _khatri_rao(h, m3, NE3, HID)))
    s3 = _mm(w32[0], _khatri_rao(h, m3, NE3, HID))
    inds3 = _argmax0(s3, O2)

    out_ref[0, 0] = jnp.clip(inds12_raw * NE2 + inds3 - 2, 0, NE3 * NE2 - 1)


def _pack(w, b, ne, ci, co):
    wt = jnp.transpose(w.reshape(H, ne, ci, co), (0, 3, 1, 2)).reshape(
        H, co, ne * ci)
    bt = jnp.transpose(b.reshape(H, ne, co), (0, 2, 1))
    return jnp.concatenate([wt, bt], axis=2)


def kernel(x_in, c1_w0, c1_b0, c1_w1, c1_b1, c1_w2, c1_b2,
           c2_w0, c2_b0, c2_w1, c2_b1, c2_w2, c2_b2,
           c3_w0, c3_b0, c3_w1, c3_b1, c3_w2, c3_b2):
    x_t = jnp.transpose(x_in[0], (1, 0, 2))

    args = [
        x_t,
        c1_w0, c1_b0.reshape(H, HID, 1),
        c1_w1, c1_b1.reshape(H, HID, 1),
        c1_w2, c1_b2.reshape(H, O1, 1),
        _pack(c2_w0, c2_b0, NE2, CH, HID),
        _pack(c2_w1, c2_b1, NE2, HID, HID),
        _pack(c2_w2, c2_b2, NE2, HID, O2),
        _pack(c3_w0, c3_b0, NE3, CH, HID),
        _pack(c3_w1, c3_b1, NE3, HID, HID),
        _pack(c3_w2, c3_b2, NE3, HID, O2),
    ]

    in_specs = [
        pl.BlockSpec((1,) + a.shape[1:], lambda h: (h, 0, 0)) for a in args
    ]

    out = pl.pallas_call(
        _line_kernel,
        grid=(H,),
        in_specs=in_specs,
        out_specs=pl.BlockSpec((1, 1, W), lambda h: (h, 0, 0)),
        out_shape=jax.ShapeDtypeStruct((H, 1, W), jnp.int32),
        compiler_params=pltpu.CompilerParams(
            dimension_semantics=("arbitrary",),
        ),
    )(*args)

    return out.reshape(1, 1, H, W)


# speedup vs baseline: 1.0080x; 1.0080x over previous
import jax
import jax.numpy as jnp
from jax.experimental import pallas as pl
from jax.experimental.pallas import tpu as pltpu

H, CH, W = 128, 64, 256
NE2 = 8
NE3 = 64
O1 = 8
O2 = 12
HID = 32


def _leaky(x):
    return jnp.where(x > 0, x, 0.01 * x)


def _argmax0(a, n):
    mx = jnp.max(a, axis=0)
    iota = jax.lax.broadcasted_iota(jnp.int32, a.shape, 0)
    cand = jnp.where(a == mx[None, :], iota, n)
    return jnp.min(cand, axis=0).astype(jnp.int32)


def _mm0(a, b):
    return jax.lax.dot_general(
        a, b, (((0,), (0,)), ((), ())), preferred_element_type=jnp.float32)


def _routed(w, b, m, hm):
    ne, ci, co = w.shape
    return _mm0(w.reshape(ne * ci, co), hm) + _mm0(b, m)


def _khatri_rao(h, m, ne, d):
    return (h[None] * m[:, None, :]).reshape(ne * d, W)


def _line_kernel(x_ref, w10, b10, w11, b11, w12, b12,
                 w20, b20, w21, b21, w22, b22,
                 w30, b30, w31, b31, w32, b32, out_ref):
    X = x_ref[0]

    h = _leaky(jnp.dot(w10[0], X, preferred_element_type=jnp.float32) + b10[0])
    h = _leaky(jnp.dot(w11[0], h, preferred_element_type=jnp.float32) + b11[0])
    s1 = jnp.dot(w12[0], h, preferred_element_type=jnp.float32) + b12[0]
    inds1 = _argmax0(s1, O1)

    e2 = jax.lax.broadcasted_iota(jnp.int32, (NE2, W), 0)
    m2 = (e2 == inds1[None, :]).astype(jnp.float32)
    h = _leaky(_routed(w20[:], b20[:], m2, _khatri_rao(X, m2, NE2, CH)))
    h = _leaky(_routed(w21[:], b21[:], m2, _khatri_rao(h, m2, NE2, HID)))
    s2 = _routed(w22[:], b22[:], m2, _khatri_rao(h, m2, NE2, HID))
    inds2 = _argmax0(s2, O2)

    inds12_raw = inds1 * NE2 + inds2 - 2
    inds12 = jnp.clip(inds12_raw, 0, NE3 - 1)

    e3 = jax.lax.broadcasted_iota(jnp.int32, (NE3, W), 0)
    m3 = (e3 == inds12[None, :]).astype(jnp.float32)
    h = _leaky(_routed(w30[:], b30[:], m3, _khatri_rao(X, m3, NE3, CH)))
    h = _leaky(_routed(w31[:], b31[:], m3, _khatri_rao(h, m3, NE3, HID)))
    s3 = _routed(w32[:], b32[:], m3, _khatri_rao(h, m3, NE3, HID))
    inds3 = _argmax0(s3, O2)

    out_ref[0, 0] = jnp.clip(inds12_raw * NE2 + inds3 - 2, 0, NE3 * NE2 - 1)


def kernel(x_in, c1_w0, c1_b0, c1_w1, c1_b1, c1_w2, c1_b2,
           c2_w0, c2_b0, c2_w1, c2_b1, c2_w2, c2_b2,
           c3_w0, c3_b0, c3_w1, c3_b1, c3_w2, c3_b2):
    x3 = jnp.transpose(x_in[0], (1, 0, 2))

    def s_x():
        return pl.BlockSpec((1, CH, W), lambda h: (h, 0, 0))

    def s1w(shape):
        return pl.BlockSpec((1,) + shape, lambda h: (h, 0, 0))

    def s1b(o):
        return pl.BlockSpec((1, o, 1), lambda h: (h, 0, 0))

    def sw(ne, ci, co):
        return pl.BlockSpec((ne, ci, co), lambda h: (h, 0, 0))

    def sb(ne, co):
        return pl.BlockSpec((ne, co), lambda h: (h, 0))

    in_specs = [
        s_x(),
        s1w((HID, CH)), s1b(HID),
        s1w((HID, HID)), s1b(HID),
        s1w((O1, HID)), s1b(O1),
        sw(NE2, CH, HID), sb(NE2, HID),
        sw(NE2, HID, HID), sb(NE2, HID),
        sw(NE2, HID, O2), sb(NE2, O2),
        sw(NE3, CH, HID), sb(NE3, HID),
        sw(NE3, HID, HID), sb(NE3, HID),
        sw(NE3, HID, O2), sb(NE3, O2),
    ]

    args = [
        x3,
        c1_w0, c1_b0.reshape(H, HID, 1),
        c1_w1, c1_b1.reshape(H, HID, 1),
        c1_w2, c1_b2.reshape(H, O1, 1),
        c2_w0, c2_b0, c2_w1, c2_b1, c2_w2, c2_b2,
        c3_w0, c3_b0, c3_w1, c3_b1, c3_w2, c3_b2,
    ]

    out = pl.pallas_call(
        _line_kernel,
        grid=(H,),
        in_specs=in_specs,
        out_specs=pl.BlockSpec((1, 1, W), lambda h: (h, 0, 0)),
        out_shape=jax.ShapeDtypeStruct((H, 1, W), jnp.int32),
        compiler_params=pltpu.CompilerParams(
            dimension_semantics=("arbitrary",),
        ),
    )(*args)

    return out.reshape(1, 1, H, W)


# speedup vs baseline: 1.0488x; 1.0405x over previous
import jax
import jax.numpy as jnp
from jax.experimental import pallas as pl
from jax.experimental.pallas import tpu as pltpu

H, CH, W = 128, 64, 256
NE2 = 8
NE3 = 64
O1 = 8
O2 = 12
HID = 32
LPB = 2


def _leaky(x):
    return jnp.where(x > 0, x, 0.01 * x)


def _argmax0(a, n):
    mx = jnp.max(a, axis=0)
    iota = jax.lax.broadcasted_iota(jnp.int32, a.shape, 0)
    cand = jnp.where(a == mx[None, :], iota, n)
    return jnp.min(cand, axis=0).astype(jnp.int32)


def _mm0(a, b):
    return jax.lax.dot_general(
        a, b, (((0,), (0,)), ((), ())), preferred_element_type=jnp.float32)


def _routed(w, b, m, hm):
    ne, ci, co = w.shape
    return _mm0(w.reshape(ne * ci, co), hm) + _mm0(b, m)


def _khatri_rao(h, m, ne, d):
    return (h[None] * m[:, None, :]).reshape(ne * d, W)


def _line_kernel(x_ref, w10, b10, w11, b11, w12, b12,
                 w20, b20, w21, b21, w22, b22,
                 w30, b30, w31, b31, w32, b32, out_ref):
    for j in range(LPB):
        _one_line(j, x_ref, w10, b10, w11, b11, w12, b12,
                  w20, b20, w21, b21, w22, b22,
                  w30, b30, w31, b31, w32, b32, out_ref)


def _one_line(j, x_ref, w10, b10, w11, b11, w12, b12,
              w20, b20, w21, b21, w22, b22,
              w30, b30, w31, b31, w32, b32, out_ref):
    X = x_ref[j]

    h = _leaky(jnp.dot(w10[j], X, preferred_element_type=jnp.float32) + b10[j])
    h = _leaky(jnp.dot(w11[j], h, preferred_element_type=jnp.float32) + b11[j])
    s1 = jnp.dot(w12[j], h, preferred_element_type=jnp.float32) + b12[j]
    inds1 = _argmax0(s1, O1)

    e2 = jax.lax.broadcasted_iota(jnp.int32, (NE2, W), 0)
    m2 = (e2 == inds1[None, :]).astype(jnp.float32)
    sl2 = pl.ds(j * NE2, NE2)
    h = _leaky(_routed(w20[sl2], b20[sl2], m2, _khatri_rao(X, m2, NE2, CH)))
    h = _leaky(_routed(w21[sl2], b21[sl2], m2, _khatri_rao(h, m2, NE2, HID)))
    s2 = _routed(w22[sl2], b22[sl2], m2, _khatri_rao(h, m2, NE2, HID))
    inds2 = _argmax0(s2, O2)

    inds12_raw = inds1 * NE2 + inds2 - 2
    inds12 = jnp.clip(inds12_raw, 0, NE3 - 1)

    e3 = jax.lax.broadcasted_iota(jnp.int32, (NE3, W), 0)
    m3 = (e3 == inds12[None, :]).astype(jnp.float32)
    sl3 = pl.ds(j * NE3, NE3)
    h = _leaky(_routed(w30[sl3], b30[sl3], m3, _khatri_rao(X, m3, NE3, CH)))
    h = _leaky(_routed(w31[sl3], b31[sl3], m3, _khatri_rao(h, m3, NE3, HID)))
    s3 = _routed(w32[sl3], b32[sl3], m3, _khatri_rao(h, m3, NE3, HID))
    inds3 = _argmax0(s3, O2)

    out_ref[j, 0] = jnp.clip(inds12_raw * NE2 + inds3 - 2, 0, NE3 * NE2 - 1)


def kernel(x_in, c1_w0, c1_b0, c1_w1, c1_b1, c1_w2, c1_b2,
           c2_w0, c2_b0, c2_w1, c2_b1, c2_w2, c2_b2,
           c3_w0, c3_b0, c3_w1, c3_b1, c3_w2, c3_b2):
    x3 = jnp.transpose(x_in[0], (1, 0, 2))

    def s_x():
        return pl.BlockSpec((LPB, CH, W), lambda h: (h, 0, 0))

    def s1w(shape):
        return pl.BlockSpec((LPB,) + shape, lambda h: (h, 0, 0))

    def s1b(o):
        return pl.BlockSpec((LPB, o, 1), lambda h: (h, 0, 0))

    def sw(ne, ci, co):
        return pl.BlockSpec((LPB * ne, ci, co), lambda h: (h, 0, 0))

    def sb(ne, co):
        return pl.BlockSpec((LPB * ne, co), lambda h: (h, 0))

    in_specs = [
        s_x(),
        s1w((HID, CH)), s1b(HID),
        s1w((HID, HID)), s1b(HID),
        s1w((O1, HID)), s1b(O1),
        sw(NE2, CH, HID), sb(NE2, HID),
        sw(NE2, HID, HID), sb(NE2, HID),
        sw(NE2, HID, O2), sb(NE2, O2),
        sw(NE3, CH, HID), sb(NE3, HID),
        sw(NE3, HID, HID), sb(NE3, HID),
        sw(NE3, HID, O2), sb(NE3, O2),
    ]

    args = [
        x3,
        c1_w0, c1_b0.reshape(H, HID, 1),
        c1_w1, c1_b1.reshape(H, HID, 1),
        c1_w2, c1_b2.reshape(H, O1, 1),
        c2_w0, c2_b0, c2_w1, c2_b1, c2_w2, c2_b2,
        c3_w0, c3_b0, c3_w1, c3_b1, c3_w2, c3_b2,
    ]

    out = pl.pallas_call(
        _line_kernel,
        grid=(H // LPB,),
        in_specs=in_specs,
        out_specs=pl.BlockSpec((LPB, 1, W), lambda h: (h, 0, 0)),
        out_shape=jax.ShapeDtypeStruct((H, 1, W), jnp.int32),
        compiler_params=pltpu.CompilerParams(
            dimension_semantics=("arbitrary",),
        ),
    )(*args)

    return out.reshape(1, 1, H, W)
